# baseline (speedup 1.0000x reference)
import functools

import jax
import jax.numpy as jnp
import numpy as np
from jax import lax
from jax.experimental import pallas as pl
from jax.experimental.pallas import tpu as pltpu
from jax.experimental.pallas import tpu_sc as plsc

B = 4096
NEWS_NUM = 5
TITLE_LEN = 30
ENTITY_LEN = 5
UC, US, UN = 8, 4, 5
USR_COL0 = NEWS_NUM * 5 + UC + UC * US
DATA_W = 225

VOCAB = 100000
COMB_W = 48

NC, NS = 2, 16
NW = NC * NS
L = 16

N_TGT = B * NEWS_NUM
TGT_PER_W = N_TGT // NW
B_PER_W = B // NW

CHUNK = 128
T_WORDS = CHUNK * TITLE_LEN
E_WORDS = CHUNK * ENTITY_LEN
TGT_CHUNKS = TGT_PER_W // CHUNK

UCHUNK = 128
U_CHUNKS = UC * NEWS_NUM * US
NSLOT = 4

BUILD_CHUNK = 400
BUILD_CHUNKS_TOTAL = VOCAB // BUILD_CHUNK

_mesh = plsc.VectorSubcoreMesh(core_axis_name="c", subcore_axis_name="s")
_params = pltpu.CompilerParams(use_tc_tiling_on_sc=False,
                               needs_layout_passes=False)


@functools.partial(
    pl.kernel,
    out_type=jax.ShapeDtypeStruct((VOCAB, COMB_W), jnp.int32),
    mesh=_mesh,
    compiler_params=_params,
    scratch_types=[
        pltpu.VMEM((BUILD_CHUNK * TITLE_LEN,), jnp.int32),
        pltpu.VMEM((BUILD_CHUNK * ENTITY_LEN,), jnp.int32),
        pltpu.VMEM((BUILD_CHUNK, COMB_W), jnp.int32),
    ],
)
def _build_kernel(title_hbm, ent_hbm, comb_out, title_v, ent_v, comb_v):
    wid = lax.axis_index("s") * NC + lax.axis_index("c")
    nchunks = jnp.where(wid < BUILD_CHUNKS_TOTAL % NW, 1 + BUILD_CHUNKS_TOTAL // NW,
                        BUILD_CHUNKS_TOTAL // NW)

    lane = lax.iota(jnp.int32, L)
    b_tcol = jnp.minimum(16 + lane, TITLE_LEN - 1)
    b_ecol = jnp.clip(lane - 14, 0, ENTITY_LEN - 1)
    b_is_t = lane < 14
    c_ecol = jnp.minimum(2 + lane, ENTITY_LEN - 1)
    c_dcol = jnp.minimum(32 + lane, COMB_W - 1)
    c_mask = lane < 3

    def chunk_body(k, carry):
        cid = wid + k * NW
        pltpu.sync_copy(title_hbm.at[pl.ds(cid * BUILD_CHUNK * TITLE_LEN,
                                           BUILD_CHUNK * TITLE_LEN)], title_v)
        pltpu.sync_copy(ent_hbm.at[pl.ds(cid * BUILD_CHUNK * ENTITY_LEN,
                                         BUILD_CHUNK * ENTITY_LEN)], ent_v)

        def row_body(r, c2):
            rowv = jnp.full((L,), r, jnp.int32)
            t_base = r * TITLE_LEN
            e_base = r * ENTITY_LEN
            a_vals = plsc.load_gather(title_v, [t_base + lane])
            plsc.store_scatter(comb_v, [rowv, lane], a_vals)
            b_t = plsc.load_gather(title_v, [t_base + b_tcol])
            b_e = plsc.load_gather(ent_v, [e_base + b_ecol])
            plsc.store_scatter(comb_v, [rowv, 16 + lane],
                               jnp.where(b_is_t, b_t, b_e))
            c_e = plsc.load_gather(ent_v, [e_base + c_ecol])
            plsc.store_scatter(comb_v, [rowv, c_dcol], c_e, mask=c_mask)
            return c2
        lax.fori_loop(0, BUILD_CHUNK, row_body, 0)

        pltpu.sync_copy(comb_v, comb_out.at[pl.ds(cid * BUILD_CHUNK, BUILD_CHUNK)])
        return carry
    lax.fori_loop(0, nchunks, chunk_body, 0)


@functools.partial(
    pl.kernel,
    out_type=[
        jax.ShapeDtypeStruct((N_TGT * TITLE_LEN,), jnp.int32),
        jax.ShapeDtypeStruct((N_TGT * ENTITY_LEN,), jnp.int32),
        jax.ShapeDtypeStruct((UC * NEWS_NUM * TITLE_LEN, NW, US, UCHUNK), jnp.int32),
        jax.ShapeDtypeStruct((UC * NEWS_NUM * ENTITY_LEN, NW, US, UCHUNK), jnp.int32),
    ],
    mesh=_mesh,
    compiler_params=_params,
    scratch_types=[
        pltpu.VMEM((B_PER_W * DATA_W,), jnp.int32),
        pltpu.VMEM((TGT_PER_W + L,), jnp.int32),
        pltpu.VMEM((U_CHUNKS * UCHUNK,), jnp.int32),
        [pltpu.VMEM((UCHUNK, COMB_W), jnp.int32) for _ in range(NSLOT)],
        [pltpu.VMEM((TITLE_LEN, UCHUNK + 1), jnp.int32) for _ in range(NSLOT)],
        [pltpu.VMEM((ENTITY_LEN, UCHUNK + 1), jnp.int32) for _ in range(NSLOT)],
        pltpu.VMEM((T_WORDS,), jnp.int32),
        pltpu.VMEM((E_WORDS + L,), jnp.int32),
        [pltpu.SemaphoreType.DMA for _ in range(NSLOT)],
        [pltpu.SemaphoreType.DMA for _ in range(NSLOT)],
    ],
)
def _gather_kernel(data_hbm, comb_hbm,
                   tgt_title_out, tgt_ent_out, usr_title_out, usr_ent_out,
                   data_v, tgt_idx_v, usr_idx_v, gbufs, uts, ues,
                   ct, ce, gsems, ssems):
    wid = lax.axis_index("s") * NC + lax.axis_index("c")
    pltpu.sync_copy(data_hbm.at[pl.ds(wid * B_PER_W * DATA_W, B_PER_W * DATA_W)],
                    data_v)

    lane = lax.iota(jnp.int32, L)

    def tgt_extract(r, carry):
        v = data_v[pl.ds(r * DATA_W, L)]
        tgt_idx_v[pl.ds(r * NEWS_NUM, L)] = v
        return carry
    lax.fori_loop(0, B_PER_W, tgt_extract, 0)

    def usr_extract(k, carry):
        rowbase = (k * L + lane) * DATA_W
        for c8 in range(UC):
            for s4 in range(US):
                for n5 in range(NEWS_NUM):
                    col = USR_COL0 + (c8 * US + s4) * UN + n5
                    dst = ((c8 * NEWS_NUM + n5) * US + s4) * CHUNK + k * L
                    usr_idx_v[pl.ds(dst, L)] = plsc.load_gather(
                        data_v, [rowbase + col])
        return carry
    lax.fori_loop(0, B_PER_W // L, usr_extract, 0)

    tgt_t_base = wid * TGT_PER_W * TITLE_LEN
    tgt_e_base = wid * TGT_PER_W * ENTITY_LEN

    def tgt_body(j, carry):
        idx = tgt_idx_v.at[pl.ds(j * CHUNK, CHUNK)]
        gb = gbufs[0]
        pltpu.async_copy(comb_hbm.at[idx], gb.at[pl.ds(0, CHUNK)], gsems[0]).wait()

        def tcomp(i, c2):
            ct[pl.ds(i * TITLE_LEN, L)] = gb[i, pl.ds(0, L)]
            ct[pl.ds(i * TITLE_LEN + 14, L)] = gb[i, pl.ds(14, L)]
            ce[pl.ds(i * ENTITY_LEN, L)] = gb[i, pl.ds(TITLE_LEN, L)]
            return c2
        lax.fori_loop(0, CHUNK, tcomp, 0)

        pltpu.sync_copy(ct, tgt_title_out.at[pl.ds(tgt_t_base + j * T_WORDS, T_WORDS)])
        pltpu.sync_copy(ce.at[pl.ds(0, E_WORDS)],
                        tgt_ent_out.at[pl.ds(tgt_e_base + j * E_WORDS, E_WORDS)])
        return carry
    lax.fori_loop(0, TGT_CHUNKS, tgt_body, 0)

    def start_gather(c, slot):
        cc = jnp.minimum(c, U_CHUNKS - 1)
        pltpu.async_copy(comb_hbm.at[usr_idx_v.at[pl.ds(cc * UCHUNK, UCHUNK)]],
                         gbufs[slot], gsems[slot])

    def wait_gather(slot):
        pltpu.make_async_copy(comb_hbm.at[pl.ds(0, UCHUNK)], gbufs[slot],
                              gsems[slot]).wait()

    def ut_slice(q, h):
        return usr_title_out.at[pl.ds(q * TITLE_LEN, TITLE_LEN), wid, h]


    def ue_slice(q, h):
        return usr_ent_out.at[pl.ds(q * ENTITY_LEN, ENTITY_LEN), wid, h]

    def ut_src(slot):
        return uts[slot].at[pl.ds(0, TITLE_LEN), pl.ds(0, UCHUNK)]

    def ue_src(slot):
        return ues[slot].at[pl.ds(0, ENTITY_LEN), pl.ds(0, UCHUNK)]

    def drain_store_pair(slot):
        pltpu.make_async_copy(ut_src(slot), ut_slice(0, 0), ssems[slot]).wait()
        pltpu.make_async_copy(ue_src(slot), ue_slice(0, 0), ssems[slot]).wait()

    def compact_u(slot):
        gb = gbufs[slot]
        ut, ue = uts[slot], ues[slot]
        w2 = 14 + lane
        ecol = jnp.clip(lane, 0, ENTITY_LEN - 1)
        emask = lane < ENTITY_LEN

        def rbody(r, c2):
            for rr in (2 * r, 2 * r + 1):
                rv = jnp.full((L,), rr, jnp.int32)
                plsc.store_scatter(ut, [lane, rv], gb[rr, pl.ds(0, L)])
                plsc.store_scatter(ut, [w2, rv], gb[rr, pl.ds(14, L)])
                plsc.store_scatter(ue, [ecol, rv], gb[rr, pl.ds(TITLE_LEN, L)],
                                   mask=emask)
            return c2
        lax.fori_loop(0, UCHUNK // 2, rbody, 0)

    for s in range(NSLOT):
        start_gather(s, s)
        pltpu.async_copy(ut_src(s), ut_slice(0, s), ssems[s])
        pltpu.async_copy(ue_src(s), ue_slice(0, s), ssems[s])

    def usr_body(t, carry):
        for s in range(NSLOT):
            c = NSLOT * t + s
            drain_store_pair(s)
            wait_gather(s)
            compact_u(s)
            start_gather(c + NSLOT, s)
            pltpu.async_copy(ut_src(s), ut_slice(t, s), ssems[s])
            pltpu.async_copy(ue_src(s), ue_slice(t, s), ssems[s])
        return carry
    lax.fori_loop(0, U_CHUNKS // NSLOT, usr_body, 0)

    for s in range(NSLOT):
        wait_gather(s)
        drain_store_pair(s)


def kernel(data, news_title_indexes, news_entity_indexes):
    comb = _build_kernel(news_title_indexes.reshape(-1),
                         news_entity_indexes.reshape(-1))

    tgt_title, tgt_ent, ut3, ue3 = _gather_kernel(data.reshape(-1), comb)

    usr_title = (ut3.reshape(UC, NEWS_NUM, TITLE_LEN, NW, US, 128)
                 .transpose(3, 5, 0, 4, 1, 2).reshape(B, UC, US, UN, TITLE_LEN))
    usr_ent = (ue3.reshape(UC, NEWS_NUM, ENTITY_LEN, NW, US, 128)
               .transpose(3, 5, 0, 4, 1, 2).reshape(B, UC, US, UN, ENTITY_LEN))

    return (
        tgt_title.reshape(B, NEWS_NUM, TITLE_LEN),
        tgt_ent.reshape(B, NEWS_NUM, ENTITY_LEN),
        usr_title,
        usr_ent,
    )


# speedup vs baseline: 1.0169x; 1.0169x over previous
import functools

import jax
import jax.numpy as jnp
import numpy as np
from jax import lax
from jax.experimental import pallas as pl
from jax.experimental.pallas import tpu as pltpu
from jax.experimental.pallas import tpu_sc as plsc

B = 4096
NEWS_NUM = 5
TITLE_LEN = 30
ENTITY_LEN = 5
UC, US, UN = 8, 4, 5
USR_COL0 = NEWS_NUM * 5 + UC + UC * US
DATA_W = 225

VOCAB = 100000
COMB_W = 48

NC, NS = 2, 16
NW = NC * NS
L = 16

N_TGT = B * NEWS_NUM
TGT_PER_W = N_TGT // NW
B_PER_W = B // NW

CHUNK = 128
T_WORDS = CHUNK * TITLE_LEN
E_WORDS = CHUNK * ENTITY_LEN
TGT_CHUNKS = TGT_PER_W // CHUNK

UCHUNK = 256
U_CHUNKS = UC * NEWS_NUM * 2
NSLOT = 2

BUILD_CHUNK = 400
BUILD_CHUNKS_TOTAL = VOCAB // BUILD_CHUNK

_mesh = plsc.VectorSubcoreMesh(core_axis_name="c", subcore_axis_name="s")
_params = pltpu.CompilerParams(use_tc_tiling_on_sc=False,
                               needs_layout_passes=False)


@functools.partial(
    pl.kernel,
    out_type=jax.ShapeDtypeStruct((VOCAB, COMB_W), jnp.int32),
    mesh=_mesh,
    compiler_params=_params,
    scratch_types=[
        pltpu.VMEM((BUILD_CHUNK * TITLE_LEN,), jnp.int32),
        pltpu.VMEM((BUILD_CHUNK * ENTITY_LEN,), jnp.int32),
        pltpu.VMEM((BUILD_CHUNK, COMB_W), jnp.int32),
    ],
)
def _build_kernel(title_hbm, ent_hbm, comb_out, title_v, ent_v, comb_v):
    wid = lax.axis_index("s") * NC + lax.axis_index("c")
    nchunks = jnp.where(wid < BUILD_CHUNKS_TOTAL % NW, 1 + BUILD_CHUNKS_TOTAL // NW,
                        BUILD_CHUNKS_TOTAL // NW)

    lane = lax.iota(jnp.int32, L)
    b_tcol = jnp.minimum(16 + lane, TITLE_LEN - 1)
    b_ecol = jnp.clip(lane - 14, 0, ENTITY_LEN - 1)
    b_is_t = lane < 14
    c_ecol = jnp.minimum(2 + lane, ENTITY_LEN - 1)
    c_dcol = jnp.minimum(32 + lane, COMB_W - 1)
    c_mask = lane < 3

    def chunk_body(k, carry):
        cid = wid + k * NW
        pltpu.sync_copy(title_hbm.at[pl.ds(cid * BUILD_CHUNK * TITLE_LEN,
                                           BUILD_CHUNK * TITLE_LEN)], title_v)
        pltpu.sync_copy(ent_hbm.at[pl.ds(cid * BUILD_CHUNK * ENTITY_LEN,
                                         BUILD_CHUNK * ENTITY_LEN)], ent_v)

        def row_body(r, c2):
            rowv = jnp.full((L,), r, jnp.int32)
            t_base = r * TITLE_LEN
            e_base = r * ENTITY_LEN
            a_vals = plsc.load_gather(title_v, [t_base + lane])
            plsc.store_scatter(comb_v, [rowv, lane], a_vals)
            b_t = plsc.load_gather(title_v, [t_base + b_tcol])
            b_e = plsc.load_gather(ent_v, [e_base + b_ecol])
            plsc.store_scatter(comb_v, [rowv, 16 + lane],
                               jnp.where(b_is_t, b_t, b_e))
            c_e = plsc.load_gather(ent_v, [e_base + c_ecol])
            plsc.store_scatter(comb_v, [rowv, c_dcol], c_e, mask=c_mask)
            return c2
        lax.fori_loop(0, BUILD_CHUNK, row_body, 0)

        pltpu.sync_copy(comb_v, comb_out.at[pl.ds(cid * BUILD_CHUNK, BUILD_CHUNK)])
        return carry
    lax.fori_loop(0, nchunks, chunk_body, 0)


@functools.partial(
    pl.kernel,
    out_type=[
        jax.ShapeDtypeStruct((N_TGT * TITLE_LEN,), jnp.int32),
        jax.ShapeDtypeStruct((N_TGT * ENTITY_LEN,), jnp.int32),
        jax.ShapeDtypeStruct((UC * NEWS_NUM * TITLE_LEN, NW, 2, UCHUNK), jnp.int32),
        jax.ShapeDtypeStruct((UC * NEWS_NUM * ENTITY_LEN, NW, 2, UCHUNK), jnp.int32),
    ],
    mesh=_mesh,
    compiler_params=_params,
    scratch_types=[
        pltpu.VMEM((B_PER_W * DATA_W,), jnp.int32),
        pltpu.VMEM((TGT_PER_W + L,), jnp.int32),
        pltpu.VMEM((U_CHUNKS * UCHUNK,), jnp.int32),
        [pltpu.VMEM((UCHUNK, COMB_W), jnp.int32) for _ in range(NSLOT)],
        [pltpu.VMEM((TITLE_LEN, UCHUNK + 1), jnp.int32) for _ in range(NSLOT)],
        [pltpu.VMEM((ENTITY_LEN, UCHUNK + 1), jnp.int32) for _ in range(NSLOT)],
        pltpu.VMEM((T_WORDS,), jnp.int32),
        pltpu.VMEM((E_WORDS + L,), jnp.int32),
        [pltpu.SemaphoreType.DMA for _ in range(NSLOT)],
        [pltpu.SemaphoreType.DMA for _ in range(NSLOT)],
    ],
)
def _gather_kernel(data_hbm, comb_hbm,
                   tgt_title_out, tgt_ent_out, usr_title_out, usr_ent_out,
                   data_v, tgt_idx_v, usr_idx_v, gbufs, uts, ues,
                   ct, ce, gsems, ssems):
    wid = lax.axis_index("s") * NC + lax.axis_index("c")
    pltpu.sync_copy(data_hbm.at[pl.ds(wid * B_PER_W * DATA_W, B_PER_W * DATA_W)],
                    data_v)

    lane = lax.iota(jnp.int32, L)

    def tgt_extract(r, carry):
        v = data_v[pl.ds(r * DATA_W, L)]
        tgt_idx_v[pl.ds(r * NEWS_NUM, L)] = v
        return carry
    lax.fori_loop(0, B_PER_W, tgt_extract, 0)

    def usr_extract(k, carry):
        rowbase = (k * L + lane) * DATA_W
        for c8 in range(UC):
            for s4 in range(US):
                for n5 in range(NEWS_NUM):
                    col = USR_COL0 + (c8 * US + s4) * UN + n5
                    dst = ((c8 * NEWS_NUM + n5) * US + s4) * CHUNK + k * L
                    usr_idx_v[pl.ds(dst, L)] = plsc.load_gather(
                        data_v, [rowbase + col])
        return carry
    lax.fori_loop(0, B_PER_W // L, usr_extract, 0)

    tgt_t_base = wid * TGT_PER_W * TITLE_LEN
    tgt_e_base = wid * TGT_PER_W * ENTITY_LEN

    def tgt_body(j, carry):
        idx = tgt_idx_v.at[pl.ds(j * CHUNK, CHUNK)]
        gb = gbufs[0]
        pltpu.async_copy(comb_hbm.at[idx], gb.at[pl.ds(0, CHUNK)], gsems[0]).wait()

        def tcomp(i, c2):
            ct[pl.ds(i * TITLE_LEN, L)] = gb[i, pl.ds(0, L)]
            ct[pl.ds(i * TITLE_LEN + 14, L)] = gb[i, pl.ds(14, L)]
            ce[pl.ds(i * ENTITY_LEN, L)] = gb[i, pl.ds(TITLE_LEN, L)]
            return c2
        lax.fori_loop(0, CHUNK, tcomp, 0)

        pltpu.sync_copy(ct, tgt_title_out.at[pl.ds(tgt_t_base + j * T_WORDS, T_WORDS)])
        pltpu.sync_copy(ce.at[pl.ds(0, E_WORDS)],
                        tgt_ent_out.at[pl.ds(tgt_e_base + j * E_WORDS, E_WORDS)])
        return carry
    lax.fori_loop(0, TGT_CHUNKS, tgt_body, 0)

    def start_gather(c, slot):
        cc = jnp.minimum(c, U_CHUNKS - 1)
        base = cc * UCHUNK
        pltpu.async_copy(comb_hbm.at[usr_idx_v.at[pl.ds(base, CHUNK)]],
                         gbufs[slot].at[pl.ds(0, CHUNK)], gsems[slot])
        pltpu.async_copy(comb_hbm.at[usr_idx_v.at[pl.ds(base + CHUNK, CHUNK)]],
                         gbufs[slot].at[pl.ds(CHUNK, CHUNK)], gsems[slot])

    def wait_gather(slot):
        pltpu.make_async_copy(comb_hbm.at[pl.ds(0, UCHUNK)], gbufs[slot],
                              gsems[slot]).wait()

    def ut_slice(q, h):
        return usr_title_out.at[pl.ds(q * TITLE_LEN, TITLE_LEN), wid, h]


    def ue_slice(q, h):
        return usr_ent_out.at[pl.ds(q * ENTITY_LEN, ENTITY_LEN), wid, h]

    def ut_src(slot):
        return uts[slot].at[pl.ds(0, TITLE_LEN), pl.ds(0, UCHUNK)]

    def ue_src(slot):
        return ues[slot].at[pl.ds(0, ENTITY_LEN), pl.ds(0, UCHUNK)]

    def drain_store_pair(slot):
        pltpu.make_async_copy(ut_src(slot), ut_slice(0, 0), ssems[slot]).wait()
        pltpu.make_async_copy(ue_src(slot), ue_slice(0, 0), ssems[slot]).wait()

    def compact_u(slot):
        gb = gbufs[slot]
        ut, ue = uts[slot], ues[slot]
        w2 = 14 + lane
        ecol = jnp.clip(lane, 0, ENTITY_LEN - 1)
        emask = lane < ENTITY_LEN

        def rbody(r, c2):
            for rr in (2 * r, 2 * r + 1):
                rv = jnp.full((L,), rr, jnp.int32)
                plsc.store_scatter(ut, [lane, rv], gb[rr, pl.ds(0, L)])
                plsc.store_scatter(ut, [w2, rv], gb[rr, pl.ds(14, L)])
                plsc.store_scatter(ue, [ecol, rv], gb[rr, pl.ds(TITLE_LEN, L)],
                                   mask=emask)
            return c2
        lax.fori_loop(0, UCHUNK // 2, rbody, 0)

    for s in range(NSLOT):
        start_gather(s, s)
        pltpu.async_copy(ut_src(s), ut_slice(0, s), ssems[s])
        pltpu.async_copy(ue_src(s), ue_slice(0, s), ssems[s])

    def usr_body(t, carry):
        for s in range(NSLOT):
            c = NSLOT * t + s
            drain_store_pair(s)
            wait_gather(s)
            compact_u(s)
            start_gather(c + NSLOT, s)
            pltpu.async_copy(ut_src(s), ut_slice(t, s), ssems[s])
            pltpu.async_copy(ue_src(s), ue_slice(t, s), ssems[s])
        return carry
    lax.fori_loop(0, U_CHUNKS // NSLOT, usr_body, 0)

    for s in range(NSLOT):
        wait_gather(s)
        drain_store_pair(s)


def kernel(data, news_title_indexes, news_entity_indexes):
    comb = _build_kernel(news_title_indexes.reshape(-1),
                         news_entity_indexes.reshape(-1))

    tgt_title, tgt_ent, ut3, ue3 = _gather_kernel(data.reshape(-1), comb)

    usr_title = (ut3.reshape(UC, NEWS_NUM, TITLE_LEN, NW, US, 128)
                 .transpose(3, 5, 0, 4, 1, 2).reshape(B, UC, US, UN, TITLE_LEN))
    usr_ent = (ue3.reshape(UC, NEWS_NUM, ENTITY_LEN, NW, US, 128)
               .transpose(3, 5, 0, 4, 1, 2).reshape(B, UC, US, UN, ENTITY_LEN))

    return (
        tgt_title.reshape(B, NEWS_NUM, TITLE_LEN),
        tgt_ent.reshape(B, NEWS_NUM, ENTITY_LEN),
        usr_title,
        usr_ent,
    )
